# baseline (speedup 1.0000x reference)
import functools

import jax
import jax.numpy as jnp
from jax import lax
from jax.experimental import pallas as pl
from jax.experimental.pallas import tpu as pltpu
from jax.experimental.pallas import tpu_sc as plsc


_NW = 32
_C = 8
_K = 16
_LANES = 16


def _sc_gather_sum(cs3, em3, idxn, idxe):
    B, N, D = cs3.shape
    K = _K
    assert B == 2 and D % _LANES == 0
    ndj = D // _LANES
    M = B * N
    wpb = _NW // 2
    assert N % _C == 0
    gpb = N // _C
    base_g = gpb // wpb
    rem_g = gpb - base_g * wpb
    R = _C * K
    assert R <= 128

    mesh = plsc.VectorSubcoreMesh(core_axis_name="c", subcore_axis_name="s",
                                  num_cores=2, num_subcores=16)

    @functools.partial(
        pl.kernel,
        out_type=jax.ShapeDtypeStruct((M, D), jnp.float32),
        mesh=mesh,
        scratch_types=[
            pltpu.VMEM((R,), jnp.int32),
            pltpu.VMEM((R,), jnp.int32),
            pltpu.VMEM((R,), jnp.int32),
            pltpu.VMEM((R, D), jnp.float32),
            pltpu.VMEM((R, D), jnp.float32),
            pltpu.VMEM((R, D), jnp.float32),
            pltpu.VMEM((_C, D), jnp.float32),
            pltpu.VMEM_SHARED((N, D), jnp.float32),
            pltpu.SemaphoreType.DMA,
            pltpu.SemaphoreType.DMA,
            pltpu.SemaphoreType.DMA,
            pltpu.SemaphoreType.DMA,
            pltpu.SemaphoreType.DMA,
            pltpu.SemaphoreType.DMA,
            pltpu.SemaphoreType.DMA,
        ],
    )
    def gather_kernel(cs_hbm, em_hbm, idxn_hbm, idxe_hbm, act_hbm,
                      ixn, ixe0, ixe1, rn, re0, re1, ob, shtab,
                      semn, seme0, seme1, semin, semie0, semie1, semo):
        b_i = lax.axis_index("c")
        l = lax.axis_index("s")
        nch = (base_g + (l < rem_g).astype(jnp.int32)).astype(jnp.int32)
        start_g = l * base_g + jnp.minimum(l, rem_g)
        wbase = b_i * N + start_g * _C
        woff = wbase * K

        @pl.when(l == 0)
        def _():
            pltpu.sync_copy(cs_hbm.at[b_i], shtab)
        plsc.subcore_barrier()

        ixe = (ixe0, ixe1)
        re = (re0, re1)
        seme = (seme0, seme1)
        semie = (semie0, semie1)

        def idxn_src(g):
            return idxn_hbm.at[pl.ds(woff + g * R, R)]

        def idxe_src(g):
            return idxe_hbm.at[pl.ds(woff + g * R, R)]

        def fire_ngather():
            pltpu.async_copy(shtab.at[ixn], rn, semn)

        def wait_ngather():
            pltpu.make_async_copy(shtab.at[ixn], rn, semn).wait()

        def fire_egather(s):
            pltpu.async_copy(em_hbm.at[b_i].at[ixe[s]], re[s], seme[s])

        def wait_egather(s):
            pltpu.make_async_copy(em_hbm.at[b_i].at[ixe[s]], re[s], seme[s]).wait()

        def drain_out():
            pltpu.make_async_copy(
                ob, act_hbm.at[pl.ds(wbase, _C)], semo).wait()

        pltpu.sync_copy(idxn_src(0), ixn)
        fire_ngather()
        for s in (0, 1):
            pltpu.sync_copy(idxe_src(s), ixe[s])
            fire_egather(s)

        def do_chunk(g, s):
            wait_egather(s)

            @pl.when(g + 2 < nch)
            def _():
                pltpu.async_copy(idxe_src(g + 2), ixe[s], semie[s])

            @pl.when(g >= 1)
            def _():
                drain_out()

            @pl.loop(0, _C)
            def _node_e(c):
                r0 = c * K
                for j in range(ndj):
                    dsj = pl.ds(j * _LANES, _LANES)
                    acc = re[s][r0, dsj]
                    for k in range(1, K):
                        acc = acc + re[s][r0 + k, dsj]
                    ob[c, dsj] = acc

            wait_ngather()

            @pl.when(g + 1 < nch)
            def _():
                pltpu.async_copy(idxn_src(g + 1), ixn, semin)

            @pl.loop(0, _C)
            def _node_n(c):
                r0 = c * K
                for j in range(ndj):
                    dsj = pl.ds(j * _LANES, _LANES)
                    acc = ob[c, dsj]
                    for k in range(K):
                        acc = acc + rn[r0 + k, dsj]
                    ob[c, dsj] = acc

            pltpu.async_copy(
                ob, act_hbm.at[pl.ds(wbase + g * _C, _C)], semo)

            @pl.when(g + 1 < nch)
            def _():
                pltpu.make_async_copy(idxn_src(g + 1), ixn, semin).wait()
                fire_ngather()

            @pl.when(g + 2 < nch)
            def _():
                pltpu.make_async_copy(idxe_src(g + 2), ixe[s], semie[s]).wait()
                fire_egather(s)

        @pl.loop(0, (nch + 1) // 2)
        def _outer(g2):
            for j in range(2):
                g = g2 * 2 + j

                @pl.when(g < nch)
                def _():
                    do_chunk(g, j)

        drain_out()

    return gather_kernel(cs3, em3, idxn, idxe)


def _gru_body(raw_ref, c_ref, an_ref, ae_ref, rbn_ref, rbe_ref,
              wu_ref, bu_ref, wr_ref, br_ref, wh_ref, bh_ref, o_ref):
    D = raw_ref.shape[1]
    c = c_ref[:]
    cn = jnp.sum((an_ref[:] == 0).astype(jnp.float32), axis=1, keepdims=True)
    ce = jnp.sum((ae_ref[:] == 0).astype(jnp.float32), axis=1, keepdims=True)
    a = raw_ref[:] - cn * rbn_ref[0] - ce * rbe_ref[0]

    def dot(x, w):
        return lax.dot_general(x, w, (((1,), (0,)), ((), ())),
                               preferred_element_type=jnp.float32)

    wu = wu_ref[:]
    wr = wr_ref[:]
    wh = wh_ref[:]
    u = jax.nn.sigmoid(dot(a, wu[:D]) + dot(c, wu[D:]) + bu_ref[:])
    r = jax.nn.sigmoid(dot(a, wr[:D]) + dot(c, wr[D:]) + br_ref[:])
    h = jnp.tanh(dot(a, wh[:D]) + dot(r * c, wh[D:]) + bh_ref[:])
    o_ref[:] = (1.0 - u) * c + u * h


def _tc_gru(raw, cs, an2, ae2, rbn, rbe, Wu, bu, Wr, br, Wh, bh):
    M, D = raw.shape
    B = rbn.shape[0]
    K = an2.shape[1]
    RB = 2000
    assert M % RB == 0
    grid = M // RB
    blocks_pb = grid // B
    row_spec = pl.BlockSpec((RB, D), lambda i: (i, 0))
    idx_spec = pl.BlockSpec((RB, K), lambda i: (i, 0))
    rb_spec = pl.BlockSpec((1, 1, D), lambda i: (i // blocks_pb, 0, 0))
    w_spec = pl.BlockSpec((2 * D, D), lambda i: (0, 0))
    b_spec = pl.BlockSpec((1, D), lambda i: (0, 0))
    return pl.pallas_call(
        _gru_body,
        grid=(grid,),
        in_specs=[row_spec, row_spec, idx_spec, idx_spec, rb_spec, rb_spec,
                  w_spec, b_spec, w_spec, b_spec, w_spec, b_spec],
        out_specs=row_spec,
        out_shape=jax.ShapeDtypeStruct((M, D), jnp.float32),
    )(raw, cs, an2, ae2,
      rbn.reshape(B, 1, D), rbe.reshape(B, 1, D),
      Wu, bu.reshape(1, D), Wr, br.reshape(1, D), Wh, bh.reshape(1, D))


def kernel(current_state, edges_m, A_nodes, A_edges, Wu, bu, Wr, br, Wh, bh):
    B, N, D = current_state.shape
    M = B * N
    an = A_nodes.astype(jnp.int32)
    ae = A_edges.astype(jnp.int32)
    raw = _sc_gather_sum(current_state, edges_m, an.reshape(-1),
                         ae.reshape(-1))
    new_state = _tc_gru(raw, current_state.reshape(M, D),
                        an.reshape(M, _K), ae.reshape(M, _K),
                        current_state[:, 0, :], edges_m[:, 0, :],
                        Wu, bu, Wr, br, Wh, bh)
    return new_state.reshape(B, N, D)


# speedup vs baseline: 1.4445x; 1.4445x over previous
import functools

import jax
import jax.numpy as jnp
from jax import lax
from jax.experimental import pallas as pl
from jax.experimental.pallas import tpu as pltpu
from jax.experimental.pallas import tpu_sc as plsc


_NW = 32
_C = 8
_K = 16
_LANES = 16


def _sc_gather_sum(cs3, em3, idxn, idxe, batch):
    B, N, D = cs3.shape
    K = _K
    assert D % _LANES == 0
    ndj = D // _LANES
    assert N % _C == 0
    gpb = N // _C
    base_g = gpb // _NW
    rem_g = gpb - base_g * _NW
    R = _C * K
    assert R <= 128 and base_g >= 2
    boff = batch * N * K

    mesh = plsc.VectorSubcoreMesh(core_axis_name="c", subcore_axis_name="s",
                                  num_cores=2, num_subcores=16)

    @functools.partial(
        pl.kernel,
        out_type=jax.ShapeDtypeStruct((N, D), jnp.float32),
        mesh=mesh,
        scratch_types=[
            pltpu.VMEM((R,), jnp.int32),
            pltpu.VMEM((R,), jnp.int32),
            pltpu.VMEM((R,), jnp.int32),
            pltpu.VMEM((R,), jnp.int32),
            pltpu.VMEM((R, D), jnp.float32),
            pltpu.VMEM((R, D), jnp.float32),
            pltpu.VMEM((R, D), jnp.float32),
            pltpu.VMEM((R, D), jnp.float32),
            pltpu.VMEM((_C, D), jnp.float32),
            pltpu.VMEM((_C, D), jnp.float32),
            pltpu.SemaphoreType.DMA,
            pltpu.SemaphoreType.DMA,
            pltpu.SemaphoreType.DMA,
            pltpu.SemaphoreType.DMA,
            pltpu.SemaphoreType.DMA,
            pltpu.SemaphoreType.DMA,
            pltpu.SemaphoreType.DMA,
            pltpu.SemaphoreType.DMA,
        ],
        name=f"sc_gather_sum_b{batch}",
    )
    def gather_kernel(cs_hbm, em_hbm, idxn_hbm, idxe_hbm, act_hbm,
                      ixn0, ixn1, ixe0, ixe1,
                      rn0, rn1, re0, re1, ob0, ob1,
                      semn0, semn1, seme0, seme1,
                      semi0, semi1, semo0, semo1):
        l = lax.axis_index("s") * 2 + lax.axis_index("c")
        nch = (base_g + (l < rem_g).astype(jnp.int32)).astype(jnp.int32)
        start_g = l * base_g + jnp.minimum(l, rem_g)
        wbase = start_g * _C
        woff = boff + wbase * K

        ixn = (ixn0, ixn1)
        ixe = (ixe0, ixe1)
        rn = (rn0, rn1)
        re = (re0, re1)
        ob = (ob0, ob1)
        semn = (semn0, semn1)
        seme = (seme0, seme1)
        semi = (semi0, semi1)
        semo = (semo0, semo1)

        def idx_src(g, table):
            off = woff + g * R
            hbm = idxn_hbm if table == 0 else idxe_hbm
            return hbm.at[pl.ds(off, R)]

        def fire_gathers(s):
            pltpu.async_copy(cs_hbm.at[batch].at[ixn[s]], rn[s], semn[s])
            pltpu.async_copy(em_hbm.at[batch].at[ixe[s]], re[s], seme[s])

        def wait_gathers(s):
            pltpu.make_async_copy(
                cs_hbm.at[batch].at[ixn[s]], rn[s], semn[s]).wait()
            pltpu.make_async_copy(
                em_hbm.at[batch].at[ixe[s]], re[s], seme[s]).wait()

        for s in (0, 1):
            pltpu.sync_copy(idx_src(s, 0), ixn[s])
            pltpu.sync_copy(idx_src(s, 1), ixe[s])
            fire_gathers(s)

        def do_chunk(g, s):
            wait_gathers(s)

            @pl.when(g + 2 < nch)
            def _():
                pltpu.async_copy(idx_src(g + 2, 0), ixn[s], semi[s])
                pltpu.async_copy(idx_src(g + 2, 1), ixe[s], semi[s])

            @pl.when(g >= 2)
            def _():
                pltpu.make_async_copy(
                    ob[s], act_hbm.at[pl.ds(wbase + (g - 2) * _C, _C)],
                    semo[s]).wait()

            @pl.loop(0, _C)
            def _node(c):
                r0 = c * K
                acc = [jnp.zeros((_LANES,), jnp.float32) for _ in range(ndj)]
                for k in range(K):
                    for j in range(ndj):
                        acc[j] = acc[j] + rn[s][r0 + k, pl.ds(j * _LANES, _LANES)]
                for k in range(K):
                    for j in range(ndj):
                        acc[j] = acc[j] + re[s][r0 + k, pl.ds(j * _LANES, _LANES)]
                for j in range(ndj):
                    ob[s][c, pl.ds(j * _LANES, _LANES)] = acc[j]

            @pl.when(g + 2 < nch)
            def _():
                pltpu.make_async_copy(idx_src(g + 2, 0), ixn[s], semi[s]).wait()
                pltpu.make_async_copy(idx_src(g + 2, 1), ixe[s], semi[s]).wait()
                fire_gathers(s)

            pltpu.async_copy(
                ob[s], act_hbm.at[pl.ds(wbase + g * _C, _C)], semo[s])

        @pl.loop(0, (nch + 1) // 2)
        def _outer(g2):
            for j in range(2):
                g = g2 * 2 + j

                @pl.when(g < nch)
                def _():
                    do_chunk(g, j)

        g_even = ((nch - 1) // 2) * 2
        g_odd = ((nch - 2) // 2) * 2 + 1
        for s, g in ((0, g_even), (1, g_odd)):
            pltpu.make_async_copy(
                ob[s], act_hbm.at[pl.ds(wbase + g * _C, _C)], semo[s]).wait()

    return gather_kernel(cs3, em3, idxn, idxe)


def _gru_body(raw_ref, c_ref, an_ref, ae_ref, rbn_ref, rbe_ref,
              wu_ref, bu_ref, wr_ref, br_ref, wh_ref, bh_ref, o_ref):
    D = raw_ref.shape[1]
    c = c_ref[0]
    an = an_ref[0]
    ae = ae_ref[0]
    cn = jnp.sum((an == 0).astype(jnp.float32), axis=1, keepdims=True)
    ce = jnp.sum((ae == 0).astype(jnp.float32), axis=1, keepdims=True)
    a = raw_ref[:] - cn * rbn_ref[0] - ce * rbe_ref[0]

    def dot(x, w):
        return lax.dot_general(x, w, (((1,), (0,)), ((), ())),
                               preferred_element_type=jnp.float32)

    wu = wu_ref[:]
    wr = wr_ref[:]
    wh = wh_ref[:]
    u = jax.nn.sigmoid(dot(a, wu[:D]) + dot(c, wu[D:]) + bu_ref[:])
    r = jax.nn.sigmoid(dot(a, wr[:D]) + dot(c, wr[D:]) + br_ref[:])
    h = jnp.tanh(dot(a, wh[:D]) + dot(r * c, wh[D:]) + bh_ref[:])
    o_ref[:] = (1.0 - u) * c + u * h


def _tc_gru(raw, cs3, an3, ae3, rbn, rbe, Wu, bu, Wr, br, Wh, bh, batch):
    N, D = raw.shape
    K = an3.shape[2]
    B = cs3.shape[0]
    RB = 2000
    assert N % RB == 0
    grid = N // RB
    raw_spec = pl.BlockSpec((RB, D), lambda i: (i, 0))
    cs_spec = pl.BlockSpec((1, RB, D), lambda i: (batch, i, 0))
    idx_spec = pl.BlockSpec((1, RB, K), lambda i: (batch, i, 0))
    rbn_spec = pl.BlockSpec((1, 1, D), lambda i: (batch, 0, 0))
    w_spec = pl.BlockSpec((2 * D, D), lambda i: (0, 0))
    b_spec = pl.BlockSpec((1, D), lambda i: (0, 0))
    return pl.pallas_call(
        _gru_body,
        grid=(grid,),
        in_specs=[raw_spec, cs_spec, idx_spec, idx_spec, rbn_spec, rbn_spec,
                  w_spec, b_spec, w_spec, b_spec, w_spec, b_spec],
        out_specs=raw_spec,
        out_shape=jax.ShapeDtypeStruct((N, D), jnp.float32),
        name=f"tc_gru_b{batch}",
    )(raw, cs3, an3, ae3, rbn.reshape(B, 1, D), rbe.reshape(B, 1, D),
      Wu, bu.reshape(1, D), Wr, br.reshape(1, D), Wh, bh.reshape(1, D))


def kernel(current_state, edges_m, A_nodes, A_edges, Wu, bu, Wr, br, Wh, bh):
    B, N, D = current_state.shape
    an = A_nodes.astype(jnp.int32)
    ae = A_edges.astype(jnp.int32)
    idxn = an.reshape(-1)
    idxe = ae.reshape(-1)
    rbn = current_state[:, 0, :]
    rbe = edges_m[:, 0, :]
    outs = []
    for b in range(B):
        raw = _sc_gather_sum(current_state, edges_m, idxn, idxe, b)
        outs.append(_tc_gru(raw, current_state, an, ae, rbn, rbe,
                            Wu, bu, Wr, br, Wh, bh, b))
    return jnp.stack(outs)


# speedup vs baseline: 1.5057x; 1.0424x over previous
import functools

import jax
import jax.numpy as jnp
from jax import lax
from jax.experimental import pallas as pl
from jax.experimental.pallas import tpu as pltpu
from jax.experimental.pallas import tpu_sc as plsc


_NW = 32
_C = 8
_K = 16
_LANES = 16


def _sc_gather_sum(cs_flat, em_flat, idxn, idxe):
    M, D = cs_flat.shape
    K = _K
    assert D % _LANES == 0
    ndj = D // _LANES
    rows_pb = M // 2
    wpb = _NW // 2
    assert rows_pb % _C == 0
    gpb = rows_pb // _C
    base_g = gpb // wpb
    rem_g = gpb - base_g * wpb
    R = _C * K
    assert R <= 128

    mesh = plsc.VectorSubcoreMesh(core_axis_name="c", subcore_axis_name="s",
                                  num_cores=2, num_subcores=16)

    @functools.partial(
        pl.kernel,
        out_type=jax.ShapeDtypeStruct((M, D), jnp.float32),
        mesh=mesh,
        scratch_types=[
            pltpu.VMEM((R,), jnp.int32),
            pltpu.VMEM((R,), jnp.int32),
            pltpu.VMEM((R,), jnp.int32),
            pltpu.VMEM((R,), jnp.int32),
            pltpu.VMEM((R, D), jnp.float32),
            pltpu.VMEM((R, D), jnp.float32),
            pltpu.VMEM((R, D), jnp.float32),
            pltpu.VMEM((R, D), jnp.float32),
            pltpu.VMEM((_C, D), jnp.float32),
            pltpu.VMEM((_C, D), jnp.float32),
            pltpu.SemaphoreType.DMA,
            pltpu.SemaphoreType.DMA,
            pltpu.SemaphoreType.DMA,
            pltpu.SemaphoreType.DMA,
            pltpu.SemaphoreType.DMA,
            pltpu.SemaphoreType.DMA,
            pltpu.SemaphoreType.DMA,
            pltpu.SemaphoreType.DMA,
        ],
    )
    def gather_kernel(cs_hbm, em_hbm, idxn_hbm, idxe_hbm, act_hbm,
                      ixn0, ixn1, ixe0, ixe1,
                      rn0, rn1, re0, re1, ob0, ob1,
                      semn0, semn1, seme0, seme1,
                      semi0, semi1, semo0, semo1):
        wid = lax.axis_index("s") * 2 + lax.axis_index("c")
        in_b1 = wid >= wpb
        b_i = in_b1.astype(jnp.int32)
        l = wid - wpb * b_i
        nch = (base_g + (l < rem_g).astype(jnp.int32)).astype(jnp.int32)
        start_g = l * base_g + jnp.minimum(l, rem_g)
        wbase = b_i * rows_pb + start_g * _C
        woff = wbase * K

        ixn = (ixn0, ixn1)
        ixe = (ixe0, ixe1)
        rn = (rn0, rn1)
        re = (re0, re1)
        ob = (ob0, ob1)
        semn = (semn0, semn1)
        seme = (seme0, seme1)
        semi = (semi0, semi1)
        semo = (semo0, semo1)

        def idx_src(g, table):
            off = woff + g * R
            hbm = idxn_hbm if table == 0 else idxe_hbm
            return hbm.at[pl.ds(off, R)]

        def fire_gathers(s):
            pltpu.async_copy(cs_hbm.at[ixn[s]], rn[s], semn[s])
            pltpu.async_copy(em_hbm.at[ixe[s]], re[s], seme[s])

        def wait_gathers(s):
            pltpu.make_async_copy(cs_hbm.at[ixn[s]], rn[s], semn[s]).wait()
            pltpu.make_async_copy(em_hbm.at[ixe[s]], re[s], seme[s]).wait()

        for s in (0, 1):
            pltpu.sync_copy(idx_src(s, 0), ixn[s])
            pltpu.sync_copy(idx_src(s, 1), ixe[s])
            fire_gathers(s)

        def do_chunk(g, s):
            wait_gathers(s)

            @pl.when(g + 2 < nch)
            def _():
                pltpu.async_copy(idx_src(g + 2, 0), ixn[s], semi[s])
                pltpu.async_copy(idx_src(g + 2, 1), ixe[s], semi[s])

            @pl.when(g >= 2)
            def _():
                pltpu.make_async_copy(
                    ob[s], act_hbm.at[pl.ds(wbase + (g - 2) * _C, _C)],
                    semo[s]).wait()

            @pl.loop(0, _C)
            def _node(c):
                r0 = c * K
                acc = [jnp.zeros((_LANES,), jnp.float32) for _ in range(ndj)]
                for k in range(K):
                    for j in range(ndj):
                        acc[j] = acc[j] + rn[s][r0 + k, pl.ds(j * _LANES, _LANES)]
                for k in range(K):
                    for j in range(ndj):
                        acc[j] = acc[j] + re[s][r0 + k, pl.ds(j * _LANES, _LANES)]
                for j in range(ndj):
                    ob[s][c, pl.ds(j * _LANES, _LANES)] = acc[j]

            @pl.when(g + 2 < nch)
            def _():
                pltpu.make_async_copy(idx_src(g + 2, 0), ixn[s], semi[s]).wait()
                pltpu.make_async_copy(idx_src(g + 2, 1), ixe[s], semi[s]).wait()
                fire_gathers(s)

            pltpu.async_copy(
                ob[s], act_hbm.at[pl.ds(wbase + g * _C, _C)], semo[s])

        @pl.loop(0, (nch + 1) // 2)
        def _outer(g2):
            for j in range(2):
                g = g2 * 2 + j

                @pl.when(g < nch)
                def _():
                    do_chunk(g, j)

        g_even = ((nch - 1) // 2) * 2
        g_odd = ((nch - 2) // 2) * 2 + 1
        for s, g in ((0, g_even), (1, g_odd)):
            pltpu.make_async_copy(
                ob[s], act_hbm.at[pl.ds(wbase + g * _C, _C)], semo[s]).wait()

    return gather_kernel(cs_flat, em_flat, idxn, idxe)


def _gru_body(raw_ref, c_ref, cn_ref, ce_ref, rbn_ref, rbe_ref,
              wu_ref, bu_ref, wr_ref, br_ref, wh_ref, bh_ref, o_ref):
    D = raw_ref.shape[1]
    c = c_ref[:]
    cn = cn_ref[0, 0, :][:, None]
    ce = ce_ref[0, 0, :][:, None]
    a = raw_ref[:] - cn * rbn_ref[0] - ce * rbe_ref[0]

    def dot(x, w):
        return lax.dot_general(x.astype(jnp.bfloat16), w.astype(jnp.bfloat16),
                               (((1,), (0,)), ((), ())),
                               preferred_element_type=jnp.float32)

    wu = wu_ref[:]
    wr = wr_ref[:]
    wh = wh_ref[:]
    u = jax.nn.sigmoid(dot(a, wu[:D]) + dot(c, wu[D:]) + bu_ref[:])
    r = jax.nn.sigmoid(dot(a, wr[:D]) + dot(c, wr[D:]) + br_ref[:])
    h = jnp.tanh(dot(a, wh[:D]) + dot(r * c, wh[D:]) + bh_ref[:])
    o_ref[:] = (1.0 - u) * c + u * h


def _tc_gru(raw, cs, cntn, cnte, rbn, rbe, Wu, bu, Wr, br, Wh, bh):
    M, D = raw.shape
    B = rbn.shape[0]
    RB = 2000
    assert M % RB == 0
    grid = M // RB
    blocks_pb = grid // B
    row_spec = pl.BlockSpec((RB, D), lambda i: (i, 0))
    cnt_spec = pl.BlockSpec((1, 1, RB), lambda i: (i, 0, 0))
    rb_spec = pl.BlockSpec((1, 1, D), lambda i: (i // blocks_pb, 0, 0))
    w_spec = pl.BlockSpec((2 * D, D), lambda i: (0, 0))
    b_spec = pl.BlockSpec((1, D), lambda i: (0, 0))
    return pl.pallas_call(
        _gru_body,
        grid=(grid,),
        in_specs=[row_spec, row_spec, cnt_spec, cnt_spec, rb_spec, rb_spec,
                  w_spec, b_spec, w_spec, b_spec, w_spec, b_spec],
        out_specs=row_spec,
        out_shape=jax.ShapeDtypeStruct((M, D), jnp.float32),
    )(raw, cs, cntn.reshape(grid, 1, RB), cnte.reshape(grid, 1, RB),
      rbn.reshape(B, 1, D), rbe.reshape(B, 1, D),
      Wu, bu.reshape(1, D), Wr, br.reshape(1, D), Wh, bh.reshape(1, D))


def kernel(current_state, edges_m, A_nodes, A_edges, Wu, bu, Wr, br, Wh, bh):
    B, N, D = current_state.shape
    E = edges_m.shape[1]
    M = B * N
    cs_flat = current_state.reshape(M, D)
    em_flat = edges_m.reshape(B * E, D)
    offn = (jnp.arange(B, dtype=jnp.int32) * N)[:, None, None]
    offe = (jnp.arange(B, dtype=jnp.int32) * E)[:, None, None]
    an = A_nodes.astype(jnp.int32)
    ae = A_edges.astype(jnp.int32)
    idxn = (an + offn).reshape(-1)
    idxe = (ae + offe).reshape(-1)
    cntn = jnp.sum((an == 0).astype(jnp.float32), axis=2).reshape(M)
    cnte = jnp.sum((ae == 0).astype(jnp.float32), axis=2).reshape(M)
    raw = _sc_gather_sum(cs_flat, em_flat, idxn, idxe)
    new_state = _tc_gru(raw, cs_flat, cntn, cnte,
                        current_state[:, 0, :], edges_m[:, 0, :],
                        Wu, bu, Wr, br, Wh, bh)
    return new_state.reshape(B, N, D)


# speedup vs baseline: 1.5286x; 1.0152x over previous
import functools

import jax
import jax.numpy as jnp
from jax import lax
from jax.experimental import pallas as pl
from jax.experimental.pallas import tpu as pltpu
from jax.experimental.pallas import tpu_sc as plsc


_NW = 32
_C = 8
_K = 16
_LANES = 16


def _sc_gather_sum(cs_flat, em_flat, idxn, idxe):
    M, D = cs_flat.shape
    K = _K
    assert D % _LANES == 0
    ndj = D // _LANES
    rows_pb = M // 2
    wpb = _NW // 2
    assert rows_pb % _C == 0
    gpb = rows_pb // _C
    base_g = gpb // wpb
    rem_g = gpb - base_g * wpb
    R = _C * K
    assert R <= 128

    mesh = plsc.VectorSubcoreMesh(core_axis_name="c", subcore_axis_name="s",
                                  num_cores=2, num_subcores=16)

    @functools.partial(
        pl.kernel,
        out_type=jax.ShapeDtypeStruct((M, D), jnp.float32),
        mesh=mesh,
        scratch_types=[
            pltpu.VMEM((R,), jnp.int32),
            pltpu.VMEM((R,), jnp.int32),
            pltpu.VMEM((R,), jnp.int32),
            pltpu.VMEM((R,), jnp.int32),
            pltpu.VMEM((R, D), jnp.float32),
            pltpu.VMEM((R, D), jnp.float32),
            pltpu.VMEM((R, D), jnp.float32),
            pltpu.VMEM((R, D), jnp.float32),
            pltpu.VMEM((_C, D), jnp.float32),
            pltpu.VMEM((_C, D), jnp.float32),
            pltpu.SemaphoreType.DMA,
            pltpu.SemaphoreType.DMA,
            pltpu.SemaphoreType.DMA,
            pltpu.SemaphoreType.DMA,
            pltpu.SemaphoreType.DMA,
            pltpu.SemaphoreType.DMA,
            pltpu.SemaphoreType.DMA,
            pltpu.SemaphoreType.DMA,
        ],
    )
    def gather_kernel(cs_hbm, em_hbm, idxn_hbm, idxe_hbm, act_hbm,
                      ixn0, ixn1, ixe0, ixe1,
                      rn0, rn1, re0, re1, ob0, ob1,
                      semn0, semn1, seme0, seme1,
                      semi0, semi1, semo0, semo1):
        wid = lax.axis_index("s") * 2 + lax.axis_index("c")
        in_b1 = wid >= wpb
        b_i = in_b1.astype(jnp.int32)
        l = wid - wpb * b_i
        nch = (base_g + (l < rem_g).astype(jnp.int32)).astype(jnp.int32)
        start_g = l * base_g + jnp.minimum(l, rem_g)
        wbase = b_i * rows_pb + start_g * _C
        woff = wbase * K

        ixn = (ixn0, ixn1)
        ixe = (ixe0, ixe1)
        rn = (rn0, rn1)
        re = (re0, re1)
        ob = (ob0, ob1)
        semn = (semn0, semn1)
        seme = (seme0, seme1)
        semi = (semi0, semi1)
        semo = (semo0, semo1)

        def idx_src(g, table):
            off = woff + g * R
            hbm = idxn_hbm if table == 0 else idxe_hbm
            return hbm.at[pl.ds(off, R)]

        def fire_gathers(s):
            pltpu.async_copy(cs_hbm.at[ixn[s]], rn[s], semn[s])
            pltpu.async_copy(em_hbm.at[ixe[s]], re[s], seme[s])

        def wait_gathers(s):
            pltpu.make_async_copy(cs_hbm.at[ixn[s]], rn[s], semn[s]).wait()
            pltpu.make_async_copy(em_hbm.at[ixe[s]], re[s], seme[s]).wait()

        for s in (0, 1):
            pltpu.sync_copy(idx_src(s, 0), ixn[s])
            pltpu.sync_copy(idx_src(s, 1), ixe[s])
            fire_gathers(s)

        def do_chunk(g, s):
            wait_gathers(s)

            @pl.when(g + 2 < nch)
            def _():
                pltpu.async_copy(idx_src(g + 2, 0), ixn[s], semi[s])
                pltpu.async_copy(idx_src(g + 2, 1), ixe[s], semi[s])

            @pl.when(g >= 2)
            def _():
                pltpu.make_async_copy(
                    ob[s], act_hbm.at[pl.ds(wbase + (g - 2) * _C, _C)],
                    semo[s]).wait()

            @pl.loop(0, _C)
            def _node(c):
                r0 = c * K
                acc = [jnp.zeros((_LANES,), jnp.float32) for _ in range(ndj)]
                for k in range(K):
                    for j in range(ndj):
                        acc[j] = acc[j] + rn[s][r0 + k, pl.ds(j * _LANES, _LANES)]
                for k in range(K):
                    for j in range(ndj):
                        acc[j] = acc[j] + re[s][r0 + k, pl.ds(j * _LANES, _LANES)]
                for j in range(ndj):
                    ob[s][c, pl.ds(j * _LANES, _LANES)] = acc[j]

            @pl.when(g + 2 < nch)
            def _():
                pltpu.make_async_copy(idx_src(g + 2, 0), ixn[s], semi[s]).wait()
                pltpu.make_async_copy(idx_src(g + 2, 1), ixe[s], semi[s]).wait()
                fire_gathers(s)

            pltpu.async_copy(
                ob[s], act_hbm.at[pl.ds(wbase + g * _C, _C)], semo[s])

        @pl.loop(0, (nch + 1) // 2)
        def _outer(g2):
            for j in range(2):
                g = g2 * 2 + j

                @pl.when(g < nch)
                def _():
                    do_chunk(g, j)

        g_even = ((nch - 1) // 2) * 2
        g_odd = ((nch - 2) // 2) * 2 + 1
        for s, g in ((0, g_even), (1, g_odd)):
            pltpu.make_async_copy(
                ob[s], act_hbm.at[pl.ds(wbase + g * _C, _C)], semo[s]).wait()

    return gather_kernel(cs_flat, em_flat, idxn, idxe)


def _gru_body(raw_ref, c_ref, cn_ref, ce_ref, rbn_ref, rbe_ref,
              wu_ref, bu_ref, wr_ref, br_ref, wh_ref, bh_ref, o_ref):
    D = raw_ref.shape[1]
    c = c_ref[:]
    cn = cn_ref[0, 0, :][:, None]
    ce = ce_ref[0, 0, :][:, None]
    a = raw_ref[:] - cn * rbn_ref[0] - ce * rbe_ref[0]

    def dot(x, w):
        return lax.dot_general(x, w, (((1,), (0,)), ((), ())),
                               preferred_element_type=jnp.float32)

    wu = wu_ref[:]
    wr = wr_ref[:]
    wh = wh_ref[:]
    u = jax.nn.sigmoid(dot(a, wu[:D]) + dot(c, wu[D:]) + bu_ref[:])
    r = jax.nn.sigmoid(dot(a, wr[:D]) + dot(c, wr[D:]) + br_ref[:])
    h = jnp.tanh(dot(a, wh[:D]) + dot(r * c, wh[D:]) + bh_ref[:])
    o_ref[:] = (1.0 - u) * c + u * h


def _tc_gru(raw, cs, cntn, cnte, rbn, rbe, Wu, bu, Wr, br, Wh, bh):
    M, D = raw.shape
    B = rbn.shape[0]
    RB = 2000
    assert M % RB == 0
    grid = M // RB
    blocks_pb = grid // B
    row_spec = pl.BlockSpec((RB, D), lambda i: (i, 0))
    cnt_spec = pl.BlockSpec((1, 1, RB), lambda i: (i, 0, 0))
    rb_spec = pl.BlockSpec((1, 1, D), lambda i: (i // blocks_pb, 0, 0))
    w_spec = pl.BlockSpec((2 * D, D), lambda i: (0, 0))
    b_spec = pl.BlockSpec((1, D), lambda i: (0, 0))
    return pl.pallas_call(
        _gru_body,
        grid=(grid,),
        in_specs=[row_spec, row_spec, cnt_spec, cnt_spec, rb_spec, rb_spec,
                  w_spec, b_spec, w_spec, b_spec, w_spec, b_spec],
        out_specs=row_spec,
        out_shape=jax.ShapeDtypeStruct((M, D), jnp.float32),
    )(raw, cs, cntn.reshape(grid, 1, RB), cnte.reshape(grid, 1, RB),
      rbn.reshape(B, 1, D), rbe.reshape(B, 1, D),
      Wu, bu.reshape(1, D), Wr, br.reshape(1, D), Wh, bh.reshape(1, D))


def kernel(current_state, edges_m, A_nodes, A_edges, Wu, bu, Wr, br, Wh, bh):
    B, N, D = current_state.shape
    E = edges_m.shape[1]
    M = B * N
    cs_flat = current_state.reshape(M, D)
    em_flat = edges_m.reshape(B * E, D)
    offn = (jnp.arange(B, dtype=jnp.int32) * N)[:, None, None]
    offe = (jnp.arange(B, dtype=jnp.int32) * E)[:, None, None]
    an = A_nodes.astype(jnp.int32)
    ae = A_edges.astype(jnp.int32)
    idxn = (an + offn).reshape(-1)
    idxe = (ae + offe).reshape(-1)
    cntn = jnp.sum((an == 0).astype(jnp.float32), axis=2).reshape(M)
    cnte = jnp.sum((ae == 0).astype(jnp.float32), axis=2).reshape(M)
    raw = _sc_gather_sum(cs_flat, em_flat, idxn, idxe)
    new_state = _tc_gru(raw, cs_flat, cntn, cnte,
                        current_state[:, 0, :], edges_m[:, 0, :],
                        Wu, bu, Wr, br, Wh, bh)
    return new_state.reshape(B, N, D)
